# baseline (speedup 1.0000x reference)
import functools
import math

import jax
import jax.numpy as jnp
import numpy as np
from jax import lax
from jax.experimental import pallas as pl
from jax.experimental.pallas import tpu as pltpu
from jax.experimental.pallas import tpu_sc as plsc

_EPS = 1e-15
_CUTOFF = 5.0

_COSPI_COEFFS = tuple(
    (-1.0) ** k * math.pi ** (2 * k) / math.factorial(2 * k)
    for k in range(13))


def _w_filter(r_ijT3, wd_aug, *, n_rbf, feat, be):
    n_blocks = r_ijT3.shape[0]
    n_edges = n_blocks * be
    fh = feat // 2
    delta = _CUTOFF / (n_rbf - 1)
    coeff = -0.5 / delta**2

    def body(r_ref, wd_ref, out_ref):
        rr = r_ref[0]
        d2 = jnp.sum(rr * rr, axis=0, keepdims=True) + 3.0 * _EPS
        d = jnp.sqrt(d2)
        t = jnp.minimum(d2 * (1.0 / _CUTOFF**2), 1.0)
        cp = jnp.full_like(t, _COSPI_COEFFS[-1])
        for a in reversed(_COSPI_COEFFS[:-1]):
            cp = cp * t + a
        c = 0.5 * (cp + 1.0)
        c = jnp.where(d < _CUTOFF, c, 0.0)
        offs = lax.broadcasted_iota(jnp.int32, (n_rbf + 1, be), 0).astype(
            jnp.float32) * delta
        e = jnp.exp(coeff * (d - offs) ** 2)
        ones = jnp.ones((1, be), jnp.float32)
        e = jnp.concatenate([e[:n_rbf], ones], axis=0)
        ec = e * c
        out_ref[...] = lax.dot_general(
            ec, wd_ref[...], (((0,), (0,)), ((), ())),
            preferred_element_type=jnp.float32)

    return pl.pallas_call(
        body,
        grid=(n_blocks,),
        in_specs=[
            pl.BlockSpec((1, 3, be), lambda i: (i, 0, 0)),
            pl.BlockSpec((n_rbf + 1, feat), lambda i: (0, 0)),
        ],
        out_specs=pl.BlockSpec((be, feat), lambda i: (i, 0)),
        out_shape=jax.ShapeDtypeStruct((n_edges, feat), jnp.float32),
    )(r_ijT3, wd_aug)


def _x_lookup(z2, et_pad, *, n_atoms, feat):
    fh = feat // 2

    del fh

    def body(z_ref, et_ref, out_ref):
        z = z_ref[...]
        ids = lax.broadcasted_iota(jnp.int32, (n_atoms, feat), 1)
        onehot = (z == ids).astype(jnp.float32)
        out_ref[...] = jnp.dot(onehot, et_ref[...],
                               preferred_element_type=jnp.float32)

    return pl.pallas_call(
        body,
        grid=(1,),
        in_specs=[
            pl.BlockSpec((n_atoms, 1), lambda i: (0, 0)),
            pl.BlockSpec((feat, feat), lambda i: (0, 0)),
        ],
        out_specs=pl.BlockSpec((n_atoms, feat), lambda i: (0, 0)),
        out_shape=jax.ShapeDtypeStruct((n_atoms, feat), jnp.float32),
    )(z2, et_pad)


def _sc_aggregate(dst, src, x, w, *, n_atoms, feat):
    n_edges = dst.shape[0]
    CH = 128
    n_chunks = n_edges // CH
    NW = 32
    base_chunks = n_chunks // NW
    extra = n_chunks - base_chunks * NW
    rpt = (n_atoms // 16) // 8 * 8
    rpt_last = n_atoms - 15 * rpt
    mesh = plsc.VectorSubcoreMesh(core_axis_name="c", subcore_axis_name="s")

    @functools.partial(
        pl.kernel,
        out_type=jax.ShapeDtypeStruct((2, n_atoms, feat), jnp.float32),
        mesh=mesh,
        scratch_types=[
            pltpu.VMEM((2, CH), jnp.int32),
            pltpu.VMEM((2, CH), jnp.int32),
            pltpu.VMEM((CH, feat), jnp.float32),
            pltpu.VMEM((CH, feat), jnp.float32),
            pltpu.VMEM_SHARED((n_atoms, feat), jnp.float32),
            pltpu.SemaphoreType.DMA,
            pltpu.SemaphoreType.DMA,
            pltpu.SemaphoreType.DMA,
            pltpu.SemaphoreType.DMA,
        ],
    )
    def sc_kernel(dst_hbm, src_hbm, x_hbm, w_hbm, out_hbm,
                  dsti, srci, rows, wbuf, agg_sh, gsem, wsem, isem0, isem1):
        cid = lax.axis_index("c")
        sid = lax.axis_index("s")
        wid = cid * 16 + sid

        @pl.loop(0, CH)
        def _(r):
            for cb in range(feat // 16):
                rows[r, pl.ds(cb * 16, 16)] = jnp.zeros((16,), jnp.float32)

        nz16 = jnp.where(sid == 15, rpt_last // 16, rpt // 16)

        @pl.loop(0, nz16)
        def _(p):
            pltpu.sync_copy(rows.at[pl.ds(0, 16)],
                            agg_sh.at[pl.ds(sid * rpt + p * 16, 16)])
        plsc.subcore_barrier()

        start = wid * base_chunks + jnp.minimum(wid, extra)
        isem = (isem0, isem1)

        def issue_idx(j, p):
            base = (start + j) * CH
            pltpu.async_copy(dst_hbm.at[pl.ds(base, CH)], dsti.at[p],
                             isem[p])
            pltpu.async_copy(src_hbm.at[pl.ds(base, CH)], srci.at[p],
                             isem[p])

        def wait_idx(j, p):
            base = (start + j) * CH
            pltpu.make_async_copy(dst_hbm.at[pl.ds(base, CH)], dsti.at[p],
                                  isem[p]).wait()
            pltpu.make_async_copy(src_hbm.at[pl.ds(base, CH)], srci.at[p],
                                  isem[p]).wait()

        def issue_gather(p):
            pltpu.async_copy(x_hbm.at[dsti.at[p]], rows, gsem)

        def wait_gather(p):
            pltpu.make_async_copy(x_hbm.at[dsti.at[p]], rows, gsem).wait()

        def issue_w(j):
            base = (start + j) * CH
            pltpu.async_copy(w_hbm.at[pl.ds(base, CH)], wbuf, wsem)

        def wait_w(j):
            base = (start + j) * CH
            pltpu.make_async_copy(w_hbm.at[pl.ds(base, CH)], wbuf,
                                  wsem).wait()

        def mult():
            @pl.loop(0, CH, unroll=4)
            def _(r):
                for cb in range(feat // 16):
                    sl = pl.ds(cb * 16, 16)
                    rows[r, sl] = rows[r, sl] * wbuf[r, sl]

        def chunk_tail_sync(j, p):
            issue_idx(j, p)
            wait_idx(j, p)
            issue_gather(p)
            wait_gather(p)
            issue_w(j)
            wait_w(j)
            mult()
            pltpu.sync_copy(rows, agg_sh.at[srci.at[p]], add=True)

        NQ = base_chunks // 2
        issue_idx(0, 0)
        wait_idx(0, 0)
        issue_gather(0)
        issue_idx(1, 1)
        issue_w(0)

        @pl.loop(0, NQ)
        def _(q):
            for u in range(2):
                p = u
                j = 2 * q + u
                wait_w(j)
                wait_gather(p)
                mult()
                if u == 0:
                    issue_w(j + 1)
                else:
                    @pl.when(q < NQ - 1)
                    def _():
                        issue_w(j + 1)
                pltpu.sync_copy(rows, agg_sh.at[srci.at[p]], add=True)
                if u == 0:
                    wait_idx(j + 1, 1 - p)
                    issue_gather(1 - p)
                else:
                    @pl.when(q < NQ - 1)
                    def _():
                        wait_idx(j + 1, 1 - p)
                        issue_gather(1 - p)

                @pl.when(q < NQ - 1)
                def _():
                    issue_idx(j + 2, p)

        @pl.when(wid < extra)
        def _():
            chunk_tail_sync(base_chunks, 0)

        plsc.subcore_barrier()

        @pl.when(sid == 15)
        def _():
            pltpu.sync_copy(
                agg_sh.at[pl.ds(15 * rpt, rpt_last)],
                out_hbm.at[cid, pl.ds(15 * rpt, rpt_last)])

        @pl.when(sid != 15)
        def _():
            pltpu.sync_copy(
                agg_sh.at[pl.ds(sid * rpt, rpt)],
                out_hbm.at[cid, pl.ds(sid * rpt, rpt)])

    return sc_kernel(dst, src, x, w)


def _combine(x, aggs, W_comb, b_comb2, *, n_atoms, feat, bn):
    def body(x_ref, a_ref, wc_ref, bc_ref, out_ref):
        agg = a_ref[0] + a_ref[1]
        s = jnp.dot(x_ref[...], wc_ref[0:feat, :],
                    preferred_element_type=jnp.float32)
        s += jnp.dot(agg, wc_ref[feat:2 * feat, :],
                     preferred_element_type=jnp.float32)
        out_ref[...] = s + bc_ref[...]

    return pl.pallas_call(
        body,
        grid=(n_atoms // bn,),
        in_specs=[
            pl.BlockSpec((bn, feat), lambda i: (i, 0)),
            pl.BlockSpec((2, bn, feat), lambda i: (0, i, 0)),
            pl.BlockSpec((2 * feat, feat), lambda i: (0, 0)),
            pl.BlockSpec((1, feat), lambda i: (0, 0)),
        ],
        out_specs=pl.BlockSpec((bn, feat), lambda i: (i, 0)),
        out_shape=jax.ShapeDtypeStruct((n_atoms, feat), jnp.float32),
    )(x, aggs, W_comb, b_comb2)


def kernel(z_number, nbrs, r_ij, embed_table, W_dist, b_dist, W_comb, b_comb):
    n_atoms = z_number.shape[0]
    feat = embed_table.shape[1]
    fh = feat // 2
    n_rbf = W_dist.shape[0]

    src = nbrs[:, 0]
    dst = nbrs[:, 1]
    et_pad = jnp.pad(embed_table, ((0, feat - embed_table.shape[0]), (0, 0)))
    z2 = z_number.reshape(-1, 1).astype(jnp.int32)

    x = _x_lookup(z2, et_pad, n_atoms=n_atoms, feat=feat)
    wd_aug = jnp.concatenate([W_dist, b_dist.reshape(1, -1)], axis=0)
    be = 2000
    r_ijT3 = jnp.transpose(r_ij.T.reshape(3, -1, be), (1, 0, 2))
    w = _w_filter(r_ijT3, wd_aug, n_rbf=n_rbf, feat=feat, be=be)

    aggs = _sc_aggregate(dst, src, x, w, n_atoms=n_atoms, feat=feat)
    s = _combine(x, aggs, W_comb,
                 b_comb.reshape(1, -1), n_atoms=n_atoms, feat=feat, bn=2000)
    v = jnp.zeros((n_atoms, feat, 3), jnp.float32)
    return (s, v)


# speedup vs baseline: 1.5725x; 1.5725x over previous
import functools
import math

import jax
import jax.numpy as jnp
import numpy as np
from jax import lax
from jax.experimental import pallas as pl
from jax.experimental.pallas import tpu as pltpu
from jax.experimental.pallas import tpu_sc as plsc

_EPS = 1e-15
_CUTOFF = 5.0

_COSPI_COEFFS = tuple(
    (-1.0) ** k * math.pi ** (2 * k) / math.factorial(2 * k)
    for k in range(13))


def _w_filter(r_ijT3, wd_aug, *, n_rbf, feat, be):
    n_blocks = r_ijT3.shape[0]
    n_edges = n_blocks * be
    fh = feat // 2
    delta = _CUTOFF / (n_rbf - 1)
    coeff = -0.5 / delta**2

    def body(r_ref, wd_ref, out_ref):
        rr = r_ref[0]
        d2 = jnp.sum(rr * rr, axis=0, keepdims=True) + 3.0 * _EPS
        d = jnp.sqrt(d2)
        t = jnp.minimum(d2 * (1.0 / _CUTOFF**2), 1.0)
        cp = jnp.full_like(t, _COSPI_COEFFS[-1])
        for a in reversed(_COSPI_COEFFS[:-1]):
            cp = cp * t + a
        c = 0.5 * (cp + 1.0)
        c = jnp.where(d < _CUTOFF, c, 0.0)
        offs = lax.broadcasted_iota(jnp.int32, (n_rbf + 1, be), 0).astype(
            jnp.float32) * delta
        e = jnp.exp(coeff * (d - offs) ** 2)
        ones = jnp.ones((1, be), jnp.float32)
        e = jnp.concatenate([e[:n_rbf], ones], axis=0)
        ec = e * c
        out_ref[...] = lax.dot_general(
            ec, wd_ref[...], (((0,), (0,)), ((), ())),
            preferred_element_type=jnp.float32)

    return pl.pallas_call(
        body,
        grid=(n_blocks,),
        in_specs=[
            pl.BlockSpec((1, 3, be), lambda i: (i, 0, 0)),
            pl.BlockSpec((n_rbf + 1, feat), lambda i: (0, 0)),
        ],
        out_specs=pl.BlockSpec((be, feat), lambda i: (i, 0)),
        out_shape=jax.ShapeDtypeStruct((n_edges, feat), jnp.float32),
    )(r_ijT3, wd_aug)


def _x_lookup(z2, et_pad, *, n_atoms, feat):
    fh = feat // 2

    del fh

    def body(z_ref, et_ref, out_ref):
        z = z_ref[...]
        ids = lax.broadcasted_iota(jnp.int32, (n_atoms, feat), 1)
        onehot = (z == ids).astype(jnp.float32)
        out_ref[...] = jnp.dot(onehot, et_ref[...],
                               preferred_element_type=jnp.float32)

    return pl.pallas_call(
        body,
        grid=(1,),
        in_specs=[
            pl.BlockSpec((n_atoms, 1), lambda i: (0, 0)),
            pl.BlockSpec((feat, feat), lambda i: (0, 0)),
        ],
        out_specs=pl.BlockSpec((n_atoms, feat), lambda i: (0, 0)),
        out_shape=jax.ShapeDtypeStruct((n_atoms, feat), jnp.float32),
    )(z2, et_pad)


def _sc_aggregate(dst, src, x, w, *, n_atoms, feat):
    n_edges = dst.shape[0]
    CH = 128
    n_chunks = n_edges // CH
    NW = 32
    base_chunks = n_chunks // NW
    extra = n_chunks - base_chunks * NW
    rpt = (n_atoms // 16) // 8 * 8
    rpt_last = n_atoms - 15 * rpt
    mesh = plsc.VectorSubcoreMesh(core_axis_name="c", subcore_axis_name="s")

    @functools.partial(
        pl.kernel,
        out_type=jax.ShapeDtypeStruct((2, n_atoms, feat), jnp.float32),
        mesh=mesh,
        scratch_types=[
            pltpu.VMEM((2, CH), jnp.int32),
            pltpu.VMEM((2, CH), jnp.int32),
            pltpu.VMEM((CH, feat), jnp.float32),
            pltpu.VMEM((CH, feat), jnp.float32),
            pltpu.VMEM_SHARED((n_atoms, feat), jnp.float32),
            pltpu.SemaphoreType.DMA,
            pltpu.SemaphoreType.DMA,
            pltpu.SemaphoreType.DMA,
            pltpu.SemaphoreType.DMA,
        ],
    )
    def sc_kernel(dst_hbm, src_hbm, x_hbm, w_hbm, out_hbm,
                  dsti, srci, rows, wbuf, agg_sh, gsem, wsem, isem0, isem1):
        cid = lax.axis_index("c")
        sid = lax.axis_index("s")
        wid = cid * 16 + sid

        @pl.loop(0, CH)
        def _(r):
            for cb in range(feat // 16):
                rows[r, pl.ds(cb * 16, 16)] = jnp.zeros((16,), jnp.float32)

        nz16 = jnp.where(sid == 15, rpt_last // 16, rpt // 16)

        @pl.loop(0, nz16)
        def _(p):
            pltpu.sync_copy(rows.at[pl.ds(0, 16)],
                            agg_sh.at[pl.ds(sid * rpt + p * 16, 16)])
        plsc.subcore_barrier()

        start = wid * base_chunks + jnp.minimum(wid, extra)
        isem = (isem0, isem1)

        def issue_idx(j, p):
            base = (start + j) * CH
            pltpu.async_copy(dst_hbm.at[pl.ds(base, CH)], dsti.at[p],
                             isem[p])
            pltpu.async_copy(src_hbm.at[pl.ds(base, CH)], srci.at[p],
                             isem[p])

        def wait_idx(j, p):
            base = (start + j) * CH
            pltpu.make_async_copy(dst_hbm.at[pl.ds(base, CH)], dsti.at[p],
                                  isem[p]).wait()
            pltpu.make_async_copy(src_hbm.at[pl.ds(base, CH)], srci.at[p],
                                  isem[p]).wait()

        def issue_gather(p):
            pltpu.async_copy(x_hbm.at[dsti.at[p]], rows, gsem)

        def wait_gather(p):
            pltpu.make_async_copy(x_hbm.at[dsti.at[p]], rows, gsem).wait()

        def issue_w(j):
            base = (start + j) * CH
            pltpu.async_copy(w_hbm.at[pl.ds(base, CH)], wbuf, wsem)

        def wait_w(j):
            base = (start + j) * CH
            pltpu.make_async_copy(w_hbm.at[pl.ds(base, CH)], wbuf,
                                  wsem).wait()

        def mult():
            @pl.loop(0, CH)
            def _(r):
                for cb in range(feat // 16):
                    sl = pl.ds(cb * 16, 16)
                    wbuf[r, sl] = wbuf[r, sl] * rows[r, sl]

        def chunk_tail_sync(j, p):
            issue_idx(j, p)
            wait_idx(j, p)
            issue_gather(p)
            wait_gather(p)
            issue_w(j)
            wait_w(j)
            mult()
            pltpu.sync_copy(wbuf, agg_sh.at[srci.at[p]], add=True)

        NQ = base_chunks // 2
        issue_idx(0, 0)
        wait_idx(0, 0)
        issue_gather(0)
        issue_idx(1, 1)
        issue_w(0)

        @pl.loop(0, NQ)
        def _(q):
            for u in range(2):
                p = u
                j = 2 * q + u
                wait_gather(p)
                wait_w(j)
                mult()
                if u == 0:
                    wait_idx(j + 1, 1 - p)
                    issue_gather(1 - p)
                else:
                    @pl.when(q < NQ - 1)
                    def _():
                        wait_idx(j + 1, 1 - p)
                        issue_gather(1 - p)
                pltpu.sync_copy(wbuf, agg_sh.at[srci.at[p]], add=True)
                if u == 0:
                    issue_w(j + 1)
                else:
                    @pl.when(q < NQ - 1)
                    def _():
                        issue_w(j + 1)

                @pl.when(q < NQ - 1)
                def _():
                    issue_idx(j + 2, p)

        @pl.when(wid < extra)
        def _():
            chunk_tail_sync(base_chunks, 0)

        plsc.subcore_barrier()

        @pl.when(sid == 15)
        def _():
            pltpu.sync_copy(
                agg_sh.at[pl.ds(15 * rpt, rpt_last)],
                out_hbm.at[cid, pl.ds(15 * rpt, rpt_last)])

        @pl.when(sid != 15)
        def _():
            pltpu.sync_copy(
                agg_sh.at[pl.ds(sid * rpt, rpt)],
                out_hbm.at[cid, pl.ds(sid * rpt, rpt)])

    return sc_kernel(dst, src, x, w)


def _combine(x, aggs, W_comb, b_comb2, *, n_atoms, feat, bn):
    def body(x_ref, a_ref, wc_ref, bc_ref, out_ref):
        agg = a_ref[0] + a_ref[1]
        s = jnp.dot(x_ref[...], wc_ref[0:feat, :],
                    preferred_element_type=jnp.float32)
        s += jnp.dot(agg, wc_ref[feat:2 * feat, :],
                     preferred_element_type=jnp.float32)
        out_ref[...] = s + bc_ref[...]

    return pl.pallas_call(
        body,
        grid=(n_atoms // bn,),
        in_specs=[
            pl.BlockSpec((bn, feat), lambda i: (i, 0)),
            pl.BlockSpec((2, bn, feat), lambda i: (0, i, 0)),
            pl.BlockSpec((2 * feat, feat), lambda i: (0, 0)),
            pl.BlockSpec((1, feat), lambda i: (0, 0)),
        ],
        out_specs=pl.BlockSpec((bn, feat), lambda i: (i, 0)),
        out_shape=jax.ShapeDtypeStruct((n_atoms, feat), jnp.float32),
    )(x, aggs, W_comb, b_comb2)


def kernel(z_number, nbrs, r_ij, embed_table, W_dist, b_dist, W_comb, b_comb):
    n_atoms = z_number.shape[0]
    feat = embed_table.shape[1]
    fh = feat // 2
    n_rbf = W_dist.shape[0]

    src = nbrs[:, 0]
    dst = nbrs[:, 1]
    et_pad = jnp.pad(embed_table, ((0, feat - embed_table.shape[0]), (0, 0)))
    z2 = z_number.reshape(-1, 1).astype(jnp.int32)

    x = _x_lookup(z2, et_pad, n_atoms=n_atoms, feat=feat)
    wd_aug = jnp.concatenate([W_dist, b_dist.reshape(1, -1)], axis=0)
    be = 2000
    r_ijT3 = jnp.transpose(r_ij.T.reshape(3, -1, be), (1, 0, 2))
    w = _w_filter(r_ijT3, wd_aug, n_rbf=n_rbf, feat=feat, be=be)

    aggs = _sc_aggregate(dst, src, x, w, n_atoms=n_atoms, feat=feat)
    s = _combine(x, aggs, W_comb,
                 b_comb.reshape(1, -1), n_atoms=n_atoms, feat=feat, bn=2000)
    v = jnp.zeros((n_atoms, feat, 3), jnp.float32)
    return (s, v)


# speedup vs baseline: 1.7105x; 1.0878x over previous
import functools
import math

import jax
import jax.numpy as jnp
import numpy as np
from jax import lax
from jax.experimental import pallas as pl
from jax.experimental.pallas import tpu as pltpu
from jax.experimental.pallas import tpu_sc as plsc

_EPS = 1e-15
_CUTOFF = 5.0

_COSPI_COEFFS = tuple(
    (-1.0) ** k * math.pi ** (2 * k) / math.factorial(2 * k)
    for k in range(13))


def _w_filter(r_ijT3, wd_aug, *, n_rbf, feat, be):
    n_blocks = r_ijT3.shape[0]
    n_edges = n_blocks * be
    fh = feat // 2
    delta = _CUTOFF / (n_rbf - 1)
    coeff = -0.5 / delta**2

    def body(r_ref, wd_ref, out_ref):
        rr = r_ref[0]
        d2 = jnp.sum(rr * rr, axis=0, keepdims=True) + 3.0 * _EPS
        d = jnp.sqrt(d2)
        t = jnp.minimum(d2 * (1.0 / _CUTOFF**2), 1.0)
        cp = jnp.full_like(t, _COSPI_COEFFS[-1])
        for a in reversed(_COSPI_COEFFS[:-1]):
            cp = cp * t + a
        c = 0.5 * (cp + 1.0)
        c = jnp.where(d < _CUTOFF, c, 0.0)
        offs = lax.broadcasted_iota(jnp.int32, (n_rbf + 1, be), 0).astype(
            jnp.float32) * delta
        e = jnp.exp(coeff * (d - offs) ** 2)
        ones = jnp.ones((1, be), jnp.float32)
        e = jnp.concatenate([e[:n_rbf], ones], axis=0)
        ec = e * c
        out_ref[...] = lax.dot_general(
            ec, wd_ref[...], (((0,), (0,)), ((), ())),
            preferred_element_type=jnp.float32)

    return pl.pallas_call(
        body,
        grid=(n_blocks,),
        in_specs=[
            pl.BlockSpec((1, 3, be), lambda i: (i, 0, 0)),
            pl.BlockSpec((n_rbf + 1, feat), lambda i: (0, 0)),
        ],
        out_specs=pl.BlockSpec((be, feat), lambda i: (i, 0)),
        out_shape=jax.ShapeDtypeStruct((n_edges, feat), jnp.float32),
    )(r_ijT3, wd_aug)


def _x_lookup(z2, et_pad, *, n_atoms, feat):
    fh = feat // 2

    del fh

    def body(z_ref, et_ref, out_ref):
        z = z_ref[...]
        ids = lax.broadcasted_iota(jnp.int32, (n_atoms, feat), 1)
        onehot = (z == ids).astype(jnp.float32)
        out_ref[...] = jnp.dot(onehot, et_ref[...],
                               preferred_element_type=jnp.float32)

    return pl.pallas_call(
        body,
        grid=(1,),
        in_specs=[
            pl.BlockSpec((n_atoms, 1), lambda i: (0, 0)),
            pl.BlockSpec((feat, feat), lambda i: (0, 0)),
        ],
        out_specs=pl.BlockSpec((n_atoms, feat), lambda i: (0, 0)),
        out_shape=jax.ShapeDtypeStruct((n_atoms, feat), jnp.float32),
    )(z2, et_pad)


def _sc_aggregate(dst, src, x, w, *, n_atoms, feat):
    n_edges = dst.shape[0]
    CH = 128
    n_chunks = n_edges // CH
    NW = 32
    base_chunks = n_chunks // NW
    extra = n_chunks - base_chunks * NW
    rpt = (n_atoms // 16) // 8 * 8
    rpt_last = n_atoms - 15 * rpt
    mesh = plsc.VectorSubcoreMesh(core_axis_name="c", subcore_axis_name="s")

    @functools.partial(
        pl.kernel,
        out_type=jax.ShapeDtypeStruct((2, n_atoms, feat), jnp.float32),
        mesh=mesh,
        scratch_types=[
            pltpu.VMEM((2, CH), jnp.int32),
            pltpu.VMEM((2, CH), jnp.int32),
            pltpu.VMEM((CH, feat), jnp.float32),
            pltpu.VMEM((CH, feat), jnp.float32),
            pltpu.VMEM_SHARED((n_atoms, feat), jnp.float32),
            pltpu.SemaphoreType.DMA,
            pltpu.SemaphoreType.DMA,
            pltpu.SemaphoreType.DMA,
            pltpu.SemaphoreType.DMA,
        ],
    )
    def sc_kernel(dst_hbm, src_hbm, x_hbm, w_hbm, out_hbm,
                  dsti, srci, rows, wbuf, agg_sh, gsem, wsem, isem0, isem1):
        cid = lax.axis_index("c")
        sid = lax.axis_index("s")
        wid = cid * 16 + sid

        @pl.loop(0, CH)
        def _(r):
            for cb in range(feat // 16):
                rows[r, pl.ds(cb * 16, 16)] = jnp.zeros((16,), jnp.float32)

        nz16 = jnp.where(sid == 15, rpt_last // 16, rpt // 16)

        @pl.loop(0, nz16)
        def _(p):
            pltpu.sync_copy(rows.at[pl.ds(0, 16)],
                            agg_sh.at[pl.ds(sid * rpt + p * 16, 16)])
        plsc.subcore_barrier()

        start = wid * base_chunks + jnp.minimum(wid, extra)
        isem = (isem0, isem1)

        def issue_idx(j, p):
            base = (start + j) * CH
            pltpu.async_copy(dst_hbm.at[pl.ds(base, CH)], dsti.at[p],
                             isem[p])
            pltpu.async_copy(src_hbm.at[pl.ds(base, CH)], srci.at[p],
                             isem[p])

        def wait_idx(j, p):
            base = (start + j) * CH
            pltpu.make_async_copy(dst_hbm.at[pl.ds(base, CH)], dsti.at[p],
                                  isem[p]).wait()
            pltpu.make_async_copy(src_hbm.at[pl.ds(base, CH)], srci.at[p],
                                  isem[p]).wait()

        def issue_gather(p):
            pltpu.async_copy(x_hbm.at[dsti.at[p]], rows, gsem)

        def wait_gather(p):
            pltpu.make_async_copy(x_hbm.at[dsti.at[p]], rows, gsem).wait()

        def issue_w(j):
            base = (start + j) * CH
            pltpu.async_copy(w_hbm.at[pl.ds(base, CH)], wbuf, wsem)

        def wait_w(j):
            base = (start + j) * CH
            pltpu.make_async_copy(w_hbm.at[pl.ds(base, CH)], wbuf,
                                  wsem).wait()

        def mult():
            @pl.loop(0, CH)
            def _(r):
                for cb in range(feat // 16):
                    sl = pl.ds(cb * 16, 16)
                    wbuf[r, sl] = wbuf[r, sl] * rows[r, sl]

        def chunk_tail_sync(j, p):
            issue_idx(j, p)
            wait_idx(j, p)
            issue_gather(p)
            wait_gather(p)
            issue_w(j)
            wait_w(j)
            mult()
            pltpu.sync_copy(wbuf, agg_sh.at[srci.at[p]], add=True)

        NQ = base_chunks // 2
        issue_idx(0, 0)
        wait_idx(0, 0)
        issue_gather(0)
        issue_idx(1, 1)
        issue_w(0)

        @pl.loop(0, NQ)
        def _(q):
            for u in range(2):
                p = u
                j = 2 * q + u
                wait_gather(p)
                wait_w(j)
                mult()
                if u == 0:
                    wait_idx(j + 1, 1 - p)
                    issue_gather(1 - p)
                else:
                    @pl.when(q < NQ - 1)
                    def _():
                        wait_idx(j + 1, 1 - p)
                        issue_gather(1 - p)
                pltpu.sync_copy(wbuf, agg_sh.at[srci.at[p]], add=True)
                if u == 0:
                    issue_w(j + 1)
                else:
                    @pl.when(q < NQ - 1)
                    def _():
                        issue_w(j + 1)

                @pl.when(q < NQ - 1)
                def _():
                    issue_idx(j + 2, p)

        if base_chunks % 2 == 1:
            chunk_tail_sync(base_chunks - 1, 0)

        @pl.when(wid < extra)
        def _():
            chunk_tail_sync(base_chunks, 0)

        plsc.subcore_barrier()

        @pl.when(sid == 15)
        def _():
            pltpu.sync_copy(
                agg_sh.at[pl.ds(15 * rpt, rpt_last)],
                out_hbm.at[cid, pl.ds(15 * rpt, rpt_last)])

        @pl.when(sid != 15)
        def _():
            pltpu.sync_copy(
                agg_sh.at[pl.ds(sid * rpt, rpt)],
                out_hbm.at[cid, pl.ds(sid * rpt, rpt)])

    return sc_kernel(dst, src, x, w)


def _combine(x, aggs0, aggs1, W_comb, b_comb2, *, n_atoms, feat, bn):
    def body(x_ref, a_ref, b_ref, wc_ref, bc_ref, out_ref):
        agg = (a_ref[0] + a_ref[1]) + (b_ref[0] + b_ref[1])
        s = jnp.dot(x_ref[...], wc_ref[0:feat, :],
                    preferred_element_type=jnp.float32)
        s += jnp.dot(agg, wc_ref[feat:2 * feat, :],
                     preferred_element_type=jnp.float32)
        out_ref[...] = s + bc_ref[...]

    return pl.pallas_call(
        body,
        grid=(n_atoms // bn,),
        in_specs=[
            pl.BlockSpec((bn, feat), lambda i: (i, 0)),
            pl.BlockSpec((2, bn, feat), lambda i: (0, i, 0)),
            pl.BlockSpec((2, bn, feat), lambda i: (0, i, 0)),
            pl.BlockSpec((2 * feat, feat), lambda i: (0, 0)),
            pl.BlockSpec((1, feat), lambda i: (0, 0)),
        ],
        out_specs=pl.BlockSpec((bn, feat), lambda i: (i, 0)),
        out_shape=jax.ShapeDtypeStruct((n_atoms, feat), jnp.float32),
    )(x, aggs0, aggs1, W_comb, b_comb2)


def kernel(z_number, nbrs, r_ij, embed_table, W_dist, b_dist, W_comb, b_comb):
    n_atoms = z_number.shape[0]
    feat = embed_table.shape[1]
    fh = feat // 2
    n_rbf = W_dist.shape[0]

    src = nbrs[:, 0]
    dst = nbrs[:, 1]
    et_pad = jnp.pad(embed_table, ((0, feat - embed_table.shape[0]), (0, 0)))
    z2 = z_number.reshape(-1, 1).astype(jnp.int32)

    x = _x_lookup(z2, et_pad, n_atoms=n_atoms, feat=feat)
    wd_aug = jnp.concatenate([W_dist, b_dist.reshape(1, -1)], axis=0)
    be = 2000
    n_edges = nbrs.shape[0]
    eh = n_edges // 2
    r_ijT3a = jnp.transpose(r_ij[:eh].T.reshape(3, -1, be), (1, 0, 2))
    r_ijT3b = jnp.transpose(r_ij[eh:].T.reshape(3, -1, be), (1, 0, 2))
    w_a = _w_filter(r_ijT3a, wd_aug, n_rbf=n_rbf, feat=feat, be=be)
    w_b = _w_filter(r_ijT3b, wd_aug, n_rbf=n_rbf, feat=feat, be=be)
    w = jnp.concatenate([w_a, w_b], axis=0)

    eh = n_edges // 2
    aggs0 = _sc_aggregate(dst[:eh], src[:eh], x, w[:eh],
                          n_atoms=n_atoms, feat=feat)
    aggs1 = _sc_aggregate(dst[eh:], src[eh:], x, w[eh:],
                          n_atoms=n_atoms, feat=feat)
    s = _combine(x, aggs0, aggs1, W_comb,
                 b_comb.reshape(1, -1), n_atoms=n_atoms, feat=feat, bn=2000)
    v = jnp.zeros((n_atoms, feat, 3), jnp.float32)
    return (s, v)
